# baseline (speedup 1.0000x reference)
import functools

import jax
import jax.numpy as jnp
from jax import lax
from jax.experimental import pallas as pl
from jax.experimental.pallas import tpu as pltpu
from jax.experimental.pallas import tpu_sc as plsc

N = 10000
E = 160000
IN_CH = 256
HID = 512
OUT = 256
H = 2
G = 16

_NUM_TILES = 16
_ET = E // _NUM_TILES
_BK = 128
_EBK = 128
_ETP = ((_ET + 2 * _EBK - 1) // (2 * _EBK)) * (2 * _EBK)
_NBE = _ETP // _EBK
_NB2 = _ETP // (2 * _BK)
_EVPAD = _ETP
_RSTRIPE = 632
_NP = 10240
_DSTRIPE = _NP // _NUM_TILES


def _mm_body(x_ref, wt_ref, am_ref, xl_ref, scal_ref):
    xb = x_ref[...]
    xl = jnp.dot(xb, wt_ref[...], preferred_element_type=jnp.float32)
    xl_ref[...] = xl
    scal_ref[...] = jnp.dot(xl, am_ref[...], preferred_element_type=jnp.float32)


def _mm(x, Wt, Am, BM=1000):
    n, k = x.shape
    f = Wt.shape[1]
    return pl.pallas_call(
        _mm_body,
        grid=(n // BM,),
        in_specs=[
            pl.BlockSpec((BM, k), lambda i: (i, 0)),
            pl.BlockSpec((k, f), lambda i: (0, 0)),
            pl.BlockSpec((f, 128), lambda i: (0, 0)),
        ],
        out_specs=[
            pl.BlockSpec((BM, f), lambda i: (i, 0)),
            pl.BlockSpec((BM, 128), lambda i: (i, 0)),
        ],
        out_shape=[
            jax.ShapeDtypeStruct((n, f), jnp.float32),
            jax.ShapeDtypeStruct((n, 128), jnp.float32),
        ],
    )(x, Wt, Am)


def _gat_edge_body(C, xl_hbm, asrc_hbm, adst_hbm, batch_hbm, src_hbm, dst_hbm,
                   z2_hbm, z1_hbm, num_hbm, den_hbm,
                   facc, dacc, src_v, dst_v, ex_v,
                   ridx_v, didx_v, hs_v, hd_v, bs_v, bd_v, as_v, ad_v,
                   hs2_v, hd2_v, bs2_v, bd2_v, as2_v, ad2_v, rows_v, sem, sem2):
    head = lax.axis_index("c")
    tid = lax.axis_index("s")
    ebase = tid * _ET
    rbase = jnp.minimum(tid * _RSTRIPE, N - _RSTRIPE)
    dbase = tid * _DSTRIPE
    iota = lax.iota(jnp.int32, 16)
    zeros16 = jnp.zeros((16,), jnp.float32)
    hc = head * C

    pltpu.sync_copy(src_hbm.at[pl.ds(ebase, _ET)], src_v.at[pl.ds(0, _ET)])
    pltpu.sync_copy(dst_hbm.at[pl.ds(ebase, _ET)], dst_v.at[pl.ds(0, _ET)])
    for j in range((_ETP - _ET) // 16):
        src_v[pl.ds(_ET + 16 * j, 16)] = iota * 0
        dst_v[pl.ds(_ET + 16 * j, 16)] = iota * 0

    hoff = head * _NP

    def _fire(b, hs_r, hd_r, as_r, ad_r, bs_r, bd_r, g_sem):
        base = b * _EBK
        for j in range(_EBK // 16):
            s16 = src_v[pl.ds(base + 16 * j, 16)]
            d16 = dst_v[pl.ds(base + 16 * j, 16)]
            hs_r[pl.ds(16 * j, 16)] = s16 + hoff
            hd_r[pl.ds(16 * j, 16)] = d16 + hoff
        pltpu.async_copy(batch_hbm.at[src_v.at[pl.ds(base, _EBK)]], bs_r, g_sem)
        pltpu.async_copy(batch_hbm.at[dst_v.at[pl.ds(base, _EBK)]], bd_r, g_sem)
        pltpu.async_copy(asrc_hbm.at[hs_r], as_r, g_sem)
        pltpu.async_copy(adst_hbm.at[hd_r], ad_r, g_sem)

    def _drain(b, hs_r, hd_r, as_r, ad_r, bs_r, bd_r, g_sem):
        base = b * _EBK
        pltpu.make_async_copy(batch_hbm.at[src_v.at[pl.ds(base, _EBK)]], bs_r, g_sem).wait()
        pltpu.make_async_copy(batch_hbm.at[dst_v.at[pl.ds(base, _EBK)]], bd_r, g_sem).wait()
        pltpu.make_async_copy(asrc_hbm.at[hs_r], as_r, g_sem).wait()
        pltpu.make_async_copy(adst_hbm.at[hd_r], ad_r, g_sem).wait()

    def _exc(b, as_r, ad_r, bs_r, bd_r):
        base = b * _EBK
        for j in range(_EBK // 16):
            a = as_r[pl.ds(16 * j, 16)] + ad_r[pl.ds(16 * j, 16)]
            a = jnp.where(a > 0, a, 0.2 * a)
            eq = bs_r[pl.ds(16 * j, 16)] == bd_r[pl.ds(16 * j, 16)]
            ex_v[pl.ds(base + 16 * j, 16)] = jnp.where(eq, jnp.exp(a), 0.0)

    bufA = (hs_v, hd_v, as_v, ad_v, bs_v, bd_v, sem)
    bufB = (hs2_v, hd2_v, as2_v, ad2_v, bs2_v, bd2_v, sem2)
    _fire(0, *bufA)

    def _exb2(bb, c):
        b0 = 2 * bb
        _fire(b0 + 1, *bufB)
        _drain(b0, *bufA)
        _exc(b0, as_v, ad_v, bs_v, bd_v)
        nxt = jnp.minimum(b0 + 2, _NBE - 2)
        _fire(nxt, *bufA)
        _drain(b0 + 1, *bufB)
        _exc(b0 + 1, as2_v, ad2_v, bs2_v, bd2_v)
        return c
    lax.fori_loop(0, _NBE // 2, _exb2, 0)
    _drain(_NBE - 2, *bufA)
    for j in range((_ETP - _ET) // 16):
        ex_v[pl.ds(_ET + 16 * j, 16)] = zeros16

    def _zero_facc():
        for q in range(0, _RSTRIPE, 128):
            ln = min(128, _RSTRIPE - q)
            pltpu.sync_copy(z2_hbm.at[pl.ds(0, ln)], facc.at[pl.ds(rbase + q, ln)])
    _zero_facc()
    pltpu.sync_copy(z1_hbm.at[pl.ds(dbase, _DSTRIPE)], dacc.at[pl.ds(dbase, _DSTRIPE)])
    plsc.subcore_barrier()

    def _bld_idx(b, p, ridx_r, didx_r):
        base = b * _BK
        for j in range(_BK // 16):
            s16 = src_v[pl.ds(base + 16 * j, 16)]
            ridx_r[pl.ds(16 * j, 16)] = s16 * (H * C) + (hc + p)
            didx_r[pl.ds(16 * j, 16)] = dst_v[pl.ds(base + 16 * j, 16)]

    def _scale(b, rows_r):
        base = b * _BK
        for g in range(_BK // 16):
            w16 = ex_v[pl.ds(base + 16 * g, 16)]
            for r in range(16):
                wv = jnp.take(w16, jnp.full((16,), r, jnp.int32))
                i = 16 * g + r
                for j in range(8):
                    rows_r[i, pl.ds(16 * j, 16)] = rows_r[i, pl.ds(16 * j, 16)] * wv

    def _pass(p, c):
        def _blk(b, cc):
            _bld_idx(b, p, ridx_v, didx_v)
            pltpu.async_copy(xl_hbm.at[ridx_v], rows_v, sem).wait()
            _scale(b, rows_v)
            pltpu.sync_copy(rows_v, facc.at[didx_v], add=True)

            @pl.when(p == 0)
            def _():
                pltpu.sync_copy(ex_v.at[pl.ds(b * _BK, _BK)], dacc.at[didx_v], add=True)
            return cc
        lax.fori_loop(0, _ETP // _BK, _blk, 0)
        plsc.subcore_barrier()

        for q in range(0, _RSTRIPE, 128):
            ln = min(128, _RSTRIPE - q)
            pltpu.sync_copy(facc.at[pl.ds(rbase + q, ln)],
                            num_hbm.at[hc + p, pl.ds(rbase + q, ln)])

        @pl.when(p == 0)
        def _():
            pltpu.sync_copy(dacc.at[pl.ds(dbase, _DSTRIPE)],
                            den_hbm.at[pl.ds(head * _NP + dbase, _DSTRIPE)])
        plsc.subcore_barrier()
        _zero_facc()
        plsc.subcore_barrier()
        return c
    lax.fori_loop(0, C, _pass, 0)


def _gat_edge(xl_rows, asrcT, adstT, batch, src, dst, C):
    mesh = plsc.VectorSubcoreMesh(core_axis_name="c", subcore_axis_name="s")
    body = functools.partial(_gat_edge_body, C)
    f = pl.kernel(
        body,
        out_type=[
            jax.ShapeDtypeStruct((H * C, N, 128), jnp.float32),
            jax.ShapeDtypeStruct((H * _NP,), jnp.float32),
        ],
        mesh=mesh,
        scratch_types=[
            pltpu.VMEM_SHARED((N, 128), jnp.float32),
            pltpu.VMEM_SHARED((_NP,), jnp.float32),
            pltpu.VMEM((_EVPAD,), jnp.int32),
            pltpu.VMEM((_EVPAD,), jnp.int32),
            pltpu.VMEM((_EVPAD,), jnp.float32),
            pltpu.VMEM((_BK,), jnp.int32),
            pltpu.VMEM((_BK,), jnp.int32),
            pltpu.VMEM((_EBK,), jnp.int32),
            pltpu.VMEM((_EBK,), jnp.int32),
            pltpu.VMEM((_EBK,), jnp.int32),
            pltpu.VMEM((_EBK,), jnp.int32),
            pltpu.VMEM((_EBK,), jnp.float32),
            pltpu.VMEM((_EBK,), jnp.float32),
            pltpu.VMEM((_EBK,), jnp.int32),
            pltpu.VMEM((_EBK,), jnp.int32),
            pltpu.VMEM((_EBK,), jnp.int32),
            pltpu.VMEM((_EBK,), jnp.int32),
            pltpu.VMEM((_EBK,), jnp.float32),
            pltpu.VMEM((_EBK,), jnp.float32),
            pltpu.VMEM((_BK, 128), jnp.float32),
            pltpu.SemaphoreType.DMA,
            pltpu.SemaphoreType.DMA,
        ],
    )
    z2 = jnp.zeros((128, 128), jnp.float32)
    z1 = jnp.zeros((_NP,), jnp.float32)
    return f(xl_rows, asrcT, adstT, batch, src, dst, z2, z1)


def _ep_body(C, OC, relu, num_ref, xl_ref, scal_ref, den_ref, b_ref, out_ref):
    xl = xl_ref[...]
    scal = scal_ref[...]
    a_self = scal[:, 0:2] + scal[:, 2:4]
    a_self = jnp.where(a_self > 0, a_self, 0.2 * a_self)
    exs = jnp.exp(a_self)
    den = jnp.maximum(den_ref[...] + exs, 1e-16)
    acc = None
    for h in range(H):
        numh = jnp.concatenate([num_ref[h * C + c] for c in range(C)], axis=-1)
        xlh = xl[:, h * OC:(h + 1) * OC]
        oh = (numh + exs[:, h:h + 1] * xlh) / den[:, h:h + 1]
        acc = oh if acc is None else acc + oh
    out = acc * (1.0 / H) + b_ref[...]
    if relu:
        out = jnp.maximum(out, 0.0)
    out_ref[...] = out


def _ep(num, xl, scal, denT, b2d, C, OC, relu, BM=1000):
    body = functools.partial(_ep_body, C, OC, relu)
    return pl.pallas_call(
        body,
        grid=(N // BM,),
        in_specs=[
            pl.BlockSpec((H * C, BM, 128), lambda i: (0, i, 0)),
            pl.BlockSpec((BM, H * OC), lambda i: (i, 0)),
            pl.BlockSpec((BM, 128), lambda i: (i, 0)),
            pl.BlockSpec((BM, 2), lambda i: (i, 0)),
            pl.BlockSpec((1, OC), lambda i: (0, 0)),
        ],
        out_specs=pl.BlockSpec((BM, OC), lambda i: (i, 0)),
        out_shape=jax.ShapeDtypeStruct((N, OC), jnp.float32),
    )(num, xl, scal, denT, b2d)


def _pool_body(h_ref, batch_ref, wa_ref, ba_ref, wct_ref, bc_ref, out_ref):
    hv = h_ref[...]
    logits = jnp.sum(hv * wa_ref[...], axis=1, keepdims=True) + ba_ref[0, 0]
    b = batch_ref[...]
    gi = lax.broadcasted_iota(jnp.int32, (1, G), 1)
    oh = b == gi
    lm = jnp.max(jnp.where(oh, logits, jnp.float32(-1e30)), axis=0, keepdims=True)
    e = jnp.where(oh, jnp.exp(logits - lm), 0.0)
    s = jnp.sum(e, axis=0, keepdims=True)
    attn = e / s
    attended = lax.dot_general(attn, hv, (((0,), (0,)), ((), ())),
                               preferred_element_type=jnp.float32)
    out_ref[...] = jnp.dot(attended, wct_ref[...],
                           preferred_element_type=jnp.float32) + bc_ref[...]


def _pool(h2, batch2d, w_att, b_att, WcT, bc2d):
    return pl.pallas_call(
        _pool_body,
        grid=(1,),
        in_specs=[
            pl.BlockSpec((N, OUT), lambda i: (0, 0)),
            pl.BlockSpec((N, 1), lambda i: (0, 0)),
            pl.BlockSpec((1, OUT), lambda i: (0, 0)),
            pl.BlockSpec((1, 1), lambda i: (0, 0)),
            pl.BlockSpec((OUT, 2), lambda i: (0, 0)),
            pl.BlockSpec((1, 2), lambda i: (0, 0)),
        ],
        out_specs=pl.BlockSpec((G, 2), lambda i: (0, 0)),
        out_shape=jax.ShapeDtypeStruct((G, 2), jnp.float32),
    )(h2, batch2d, w_att, b_att, WcT, bc2d)


def _att_matrix(att_src, att_dst, OC):
    am = jnp.zeros((H * OC, 128), jnp.float32)
    asr = att_src.reshape(H, OC)
    adr = att_dst.reshape(H, OC)
    for h in range(H):
        am = am.at[h * OC:(h + 1) * OC, h].set(asr[h])
        am = am.at[h * OC:(h + 1) * OC, 2 + h].set(adr[h])
    return am


def _padT(a):
    return jnp.zeros((H, _NP), jnp.float32).at[:, :N].set(a.T).reshape(-1)


def kernel(x, edge_index, batch, W1, att_src1, att_dst1, b1,
           W2, att_src2, att_dst2, b2, w_att, b_att, Wc, bc):
    src = edge_index[0].astype(jnp.int32)
    dst = edge_index[1].astype(jnp.int32)
    batch_i = batch.astype(jnp.int32)

    C1 = HID // 128
    C2 = OUT // 128

    xl1, scal1 = _mm(x, W1.T, _att_matrix(att_src1, att_dst1, HID))
    num1, den1 = _gat_edge(xl1.reshape(N * H * C1, 128),
                           _padT(scal1[:, 0:2]), _padT(scal1[:, 2:4]),
                           batch_i, src, dst, C1)
    denT1 = den1.reshape(H, _NP)[:, :N].T
    h = _ep(num1, xl1, scal1, denT1, b1.reshape(1, HID), C1, HID, relu=True)

    xl2, scal2 = _mm(h, W2.T, _att_matrix(att_src2, att_dst2, OUT))
    num2, den2 = _gat_edge(xl2.reshape(N * H * C2, 128),
                           _padT(scal2[:, 0:2]), _padT(scal2[:, 2:4]),
                           batch_i, src, dst, C2)
    denT2 = den2.reshape(H, _NP)[:, :N].T
    h2 = _ep(num2, xl2, scal2, denT2, b2.reshape(1, OUT), C2, OUT, relu=False)

    return _pool(h2, batch_i.reshape(N, 1), w_att, b_att.reshape(1, 1), Wc.T,
                 bc.reshape(1, 2))


# speedup vs baseline: 1.0145x; 1.0145x over previous
import functools

import jax
import jax.numpy as jnp
from jax import lax
from jax.experimental import pallas as pl
from jax.experimental.pallas import tpu as pltpu
from jax.experimental.pallas import tpu_sc as plsc

N = 10000
E = 160000
IN_CH = 256
HID = 512
OUT = 256
H = 2
G = 16

_NUM_TILES = 16
_ET = E // _NUM_TILES
_BK = 128
_EBK = 128
_ETP = ((_ET + 2 * _EBK - 1) // (2 * _EBK)) * (2 * _EBK)
_NBE = _ETP // _EBK
_NB2 = _ETP // (2 * _BK)
_EVPAD = _ETP
_RSTRIPE = 632
_NP = 10240
_DSTRIPE = _NP // _NUM_TILES


def _mm_body(x_ref, wt_ref, am_ref, xl_ref, scal_ref):
    xb = x_ref[...]
    xl = jnp.dot(xb, wt_ref[...], preferred_element_type=jnp.float32)
    xl_ref[...] = xl
    scal_ref[...] = jnp.dot(xl, am_ref[...], preferred_element_type=jnp.float32)


def _mm(x, Wt, Am, BM=1000):
    n, k = x.shape
    f = Wt.shape[1]
    return pl.pallas_call(
        _mm_body,
        grid=(n // BM,),
        in_specs=[
            pl.BlockSpec((BM, k), lambda i: (i, 0)),
            pl.BlockSpec((k, f), lambda i: (0, 0)),
            pl.BlockSpec((f, 128), lambda i: (0, 0)),
        ],
        out_specs=[
            pl.BlockSpec((BM, f), lambda i: (i, 0)),
            pl.BlockSpec((BM, 128), lambda i: (i, 0)),
        ],
        out_shape=[
            jax.ShapeDtypeStruct((n, f), jnp.float32),
            jax.ShapeDtypeStruct((n, 128), jnp.float32),
        ],
    )(x, Wt, Am)


def _gat_edge_body(C, xl_hbm, asrc_hbm, adst_hbm, batch_hbm, src_hbm, dst_hbm,
                   z2_hbm, z1_hbm, num_hbm, den_hbm,
                   facc, dacc, src_v, dst_v, ex_v,
                   ridx_v, didx_v, hs_v, hd_v, bs_v, bd_v, as_v, ad_v,
                   rows_v, sem):
    head = lax.axis_index("c")
    tid = lax.axis_index("s")
    ebase = tid * _ET
    rbase = jnp.minimum(tid * _RSTRIPE, N - _RSTRIPE)
    dbase = tid * _DSTRIPE
    iota = lax.iota(jnp.int32, 16)
    zeros16 = jnp.zeros((16,), jnp.float32)
    hc = head * C

    pltpu.sync_copy(src_hbm.at[pl.ds(ebase, _ET)], src_v.at[pl.ds(0, _ET)])
    pltpu.sync_copy(dst_hbm.at[pl.ds(ebase, _ET)], dst_v.at[pl.ds(0, _ET)])
    for j in range((_ETP - _ET) // 16):
        src_v[pl.ds(_ET + 16 * j, 16)] = iota * 0
        dst_v[pl.ds(_ET + 16 * j, 16)] = iota * 0

    hoff = head * _NP
    def _exb(b, c):
        base = b * _EBK
        for j in range(_EBK // 16):
            s16 = src_v[pl.ds(base + 16 * j, 16)]
            d16 = dst_v[pl.ds(base + 16 * j, 16)]
            hs_v[pl.ds(16 * j, 16)] = s16 + hoff
            hd_v[pl.ds(16 * j, 16)] = d16 + hoff
        c1 = pltpu.async_copy(batch_hbm.at[src_v.at[pl.ds(base, _EBK)]], bs_v, sem)
        c2 = pltpu.async_copy(batch_hbm.at[dst_v.at[pl.ds(base, _EBK)]], bd_v, sem)
        c3 = pltpu.async_copy(asrc_hbm.at[hs_v], as_v, sem)
        c4 = pltpu.async_copy(adst_hbm.at[hd_v], ad_v, sem)
        c1.wait(); c2.wait(); c3.wait(); c4.wait()
        for j in range(_EBK // 16):
            a = as_v[pl.ds(16 * j, 16)] + ad_v[pl.ds(16 * j, 16)]
            a = jnp.where(a > 0, a, 0.2 * a)
            eq = bs_v[pl.ds(16 * j, 16)] == bd_v[pl.ds(16 * j, 16)]
            ex_v[pl.ds(base + 16 * j, 16)] = jnp.where(eq, jnp.exp(a), 0.0)
        return c
    lax.fori_loop(0, _NBE, _exb, 0)
    for j in range((_ETP - _ET) // 16):
        ex_v[pl.ds(_ET + 16 * j, 16)] = zeros16

    def _zero_facc():
        for q in range(0, _RSTRIPE, 128):
            ln = min(128, _RSTRIPE - q)
            pltpu.sync_copy(z2_hbm.at[pl.ds(0, ln)], facc.at[pl.ds(rbase + q, ln)])
    _zero_facc()
    pltpu.sync_copy(z1_hbm.at[pl.ds(dbase, _DSTRIPE)], dacc.at[pl.ds(dbase, _DSTRIPE)])
    plsc.subcore_barrier()

    def _bld_idx(b, p, ridx_r, didx_r):
        base = b * _BK
        for j in range(_BK // 16):
            s16 = src_v[pl.ds(base + 16 * j, 16)]
            ridx_r[pl.ds(16 * j, 16)] = s16 * (H * C) + (hc + p)
            didx_r[pl.ds(16 * j, 16)] = dst_v[pl.ds(base + 16 * j, 16)]

    def _scale(b, rows_r):
        base = b * _BK
        for g in range(_BK // 16):
            w16 = ex_v[pl.ds(base + 16 * g, 16)]
            for r in range(16):
                wv = jnp.take(w16, jnp.full((16,), r, jnp.int32))
                i = 16 * g + r
                for j in range(8):
                    rows_r[i, pl.ds(16 * j, 16)] = rows_r[i, pl.ds(16 * j, 16)] * wv

    def _pass(p, c):
        def _blk(b, cc):
            _bld_idx(b, p, ridx_v, didx_v)
            pltpu.async_copy(xl_hbm.at[ridx_v], rows_v, sem).wait()
            _scale(b, rows_v)
            pltpu.sync_copy(rows_v, facc.at[didx_v], add=True)

            @pl.when(p == 0)
            def _():
                pltpu.sync_copy(ex_v.at[pl.ds(b * _BK, _BK)], dacc.at[didx_v], add=True)
            return cc
        lax.fori_loop(0, _ETP // _BK, _blk, 0)
        plsc.subcore_barrier()

        for q in range(0, _RSTRIPE, 128):
            ln = min(128, _RSTRIPE - q)
            pltpu.sync_copy(facc.at[pl.ds(rbase + q, ln)],
                            num_hbm.at[hc + p, pl.ds(rbase + q, ln)])

        @pl.when(p == 0)
        def _():
            pltpu.sync_copy(dacc.at[pl.ds(dbase, _DSTRIPE)],
                            den_hbm.at[pl.ds(head * _NP + dbase, _DSTRIPE)])
        plsc.subcore_barrier()
        _zero_facc()
        plsc.subcore_barrier()
        return c
    lax.fori_loop(0, C, _pass, 0)


def _gat_edge(xl_rows, asrcT, adstT, batch, src, dst, C):
    mesh = plsc.VectorSubcoreMesh(core_axis_name="c", subcore_axis_name="s")
    body = functools.partial(_gat_edge_body, C)
    f = pl.kernel(
        body,
        out_type=[
            jax.ShapeDtypeStruct((H * C, N, 128), jnp.float32),
            jax.ShapeDtypeStruct((H * _NP,), jnp.float32),
        ],
        mesh=mesh,
        scratch_types=[
            pltpu.VMEM_SHARED((N, 128), jnp.float32),
            pltpu.VMEM_SHARED((_NP,), jnp.float32),
            pltpu.VMEM((_EVPAD,), jnp.int32),
            pltpu.VMEM((_EVPAD,), jnp.int32),
            pltpu.VMEM((_EVPAD,), jnp.float32),
            pltpu.VMEM((_BK,), jnp.int32),
            pltpu.VMEM((_BK,), jnp.int32),
            pltpu.VMEM((_EBK,), jnp.int32),
            pltpu.VMEM((_EBK,), jnp.int32),
            pltpu.VMEM((_EBK,), jnp.int32),
            pltpu.VMEM((_EBK,), jnp.int32),
            pltpu.VMEM((_EBK,), jnp.float32),
            pltpu.VMEM((_EBK,), jnp.float32),
            pltpu.VMEM((_BK, 128), jnp.float32),
            pltpu.SemaphoreType.DMA,
        ],
    )
    z2 = jnp.zeros((128, 128), jnp.float32)
    z1 = jnp.zeros((_NP,), jnp.float32)
    return f(xl_rows, asrcT, adstT, batch, src, dst, z2, z1)


def _ep_body(C, OC, relu, num_ref, xl_ref, scal_ref, den_ref, b_ref, out_ref):
    xl = xl_ref[...]
    scal = scal_ref[...]
    a_self = scal[:, 0:2] + scal[:, 2:4]
    a_self = jnp.where(a_self > 0, a_self, 0.2 * a_self)
    exs = jnp.exp(a_self)
    den = jnp.maximum(den_ref[...] + exs, 1e-16)
    acc = None
    for h in range(H):
        numh = jnp.concatenate([num_ref[h * C + c] for c in range(C)], axis=-1)
        xlh = xl[:, h * OC:(h + 1) * OC]
        oh = (numh + exs[:, h:h + 1] * xlh) / den[:, h:h + 1]
        acc = oh if acc is None else acc + oh
    out = acc * (1.0 / H) + b_ref[...]
    if relu:
        out = jnp.maximum(out, 0.0)
    out_ref[...] = out


def _ep(num, xl, scal, denT, b2d, C, OC, relu, BM=1000):
    body = functools.partial(_ep_body, C, OC, relu)
    return pl.pallas_call(
        body,
        grid=(N // BM,),
        in_specs=[
            pl.BlockSpec((H * C, BM, 128), lambda i: (0, i, 0)),
            pl.BlockSpec((BM, H * OC), lambda i: (i, 0)),
            pl.BlockSpec((BM, 128), lambda i: (i, 0)),
            pl.BlockSpec((BM, 2), lambda i: (i, 0)),
            pl.BlockSpec((1, OC), lambda i: (0, 0)),
        ],
        out_specs=pl.BlockSpec((BM, OC), lambda i: (i, 0)),
        out_shape=jax.ShapeDtypeStruct((N, OC), jnp.float32),
    )(num, xl, scal, denT, b2d)


def _pool_body(h_ref, batch_ref, wa_ref, ba_ref, wct_ref, bc_ref, out_ref):
    hv = h_ref[...]
    logits = jnp.sum(hv * wa_ref[...], axis=1, keepdims=True) + ba_ref[0, 0]
    b = batch_ref[...]
    gi = lax.broadcasted_iota(jnp.int32, (1, G), 1)
    oh = b == gi
    lm = jnp.max(jnp.where(oh, logits, jnp.float32(-1e30)), axis=0, keepdims=True)
    e = jnp.where(oh, jnp.exp(logits - lm), 0.0)
    s = jnp.sum(e, axis=0, keepdims=True)
    attn = e / s
    attended = lax.dot_general(attn, hv, (((0,), (0,)), ((), ())),
                               preferred_element_type=jnp.float32)
    out_ref[...] = jnp.dot(attended, wct_ref[...],
                           preferred_element_type=jnp.float32) + bc_ref[...]


def _pool(h2, batch2d, w_att, b_att, WcT, bc2d):
    return pl.pallas_call(
        _pool_body,
        grid=(1,),
        in_specs=[
            pl.BlockSpec((N, OUT), lambda i: (0, 0)),
            pl.BlockSpec((N, 1), lambda i: (0, 0)),
            pl.BlockSpec((1, OUT), lambda i: (0, 0)),
            pl.BlockSpec((1, 1), lambda i: (0, 0)),
            pl.BlockSpec((OUT, 2), lambda i: (0, 0)),
            pl.BlockSpec((1, 2), lambda i: (0, 0)),
        ],
        out_specs=pl.BlockSpec((G, 2), lambda i: (0, 0)),
        out_shape=jax.ShapeDtypeStruct((G, 2), jnp.float32),
    )(h2, batch2d, w_att, b_att, WcT, bc2d)


def _att_matrix(att_src, att_dst, OC):
    am = jnp.zeros((H * OC, 128), jnp.float32)
    asr = att_src.reshape(H, OC)
    adr = att_dst.reshape(H, OC)
    for h in range(H):
        am = am.at[h * OC:(h + 1) * OC, h].set(asr[h])
        am = am.at[h * OC:(h + 1) * OC, 2 + h].set(adr[h])
    return am


def _padT(a):
    return jnp.zeros((H, _NP), jnp.float32).at[:, :N].set(a.T).reshape(-1)


def kernel(x, edge_index, batch, W1, att_src1, att_dst1, b1,
           W2, att_src2, att_dst2, b2, w_att, b_att, Wc, bc):
    src = edge_index[0].astype(jnp.int32)
    dst = edge_index[1].astype(jnp.int32)
    batch_i = batch.astype(jnp.int32)

    C1 = HID // 128
    C2 = OUT // 128

    xl1, scal1 = _mm(x, W1.T, _att_matrix(att_src1, att_dst1, HID))
    num1, den1 = _gat_edge(xl1.reshape(N * H * C1, 128),
                           _padT(scal1[:, 0:2]), _padT(scal1[:, 2:4]),
                           batch_i, src, dst, C1)
    denT1 = den1.reshape(H, _NP)[:, :N].T
    h = _ep(num1, xl1, scal1, denT1, b1.reshape(1, HID), C1, HID, relu=True)

    xl2, scal2 = _mm(h, W2.T, _att_matrix(att_src2, att_dst2, OUT))
    num2, den2 = _gat_edge(xl2.reshape(N * H * C2, 128),
                           _padT(scal2[:, 0:2]), _padT(scal2[:, 2:4]),
                           batch_i, src, dst, C2)
    denT2 = den2.reshape(H, _NP)[:, :N].T
    h2 = _ep(num2, xl2, scal2, denT2, b2.reshape(1, OUT), C2, OUT, relu=False)

    return _pool(h2, batch_i.reshape(N, 1), w_att, b_att.reshape(1, 1), Wc.T,
                 bc.reshape(1, 2))


# speedup vs baseline: 1.2474x; 1.2296x over previous
import functools

import jax
import jax.numpy as jnp
from jax import lax
from jax.experimental import pallas as pl
from jax.experimental.pallas import tpu as pltpu
from jax.experimental.pallas import tpu_sc as plsc

N = 10000
E = 160000
IN_CH = 256
HID = 512
OUT = 256
H = 2
G = 16

_NUM_TILES = 16
_ET = E // _NUM_TILES
_BK = 128
_EBK = 128
_ETP = ((_ET + _EBK - 1) // _EBK) * _EBK
_NBE = _ETP // _EBK
_NB2 = _ETP // (2 * _BK)
_EVPAD = _ETP
_RSTRIPE = 632
_NP = 10240
_DSTRIPE = _NP // _NUM_TILES


def _mm_body(x_ref, wt_ref, am_ref, xl_ref, scal_ref):
    xb = x_ref[...]
    xl = jnp.dot(xb, wt_ref[...], preferred_element_type=jnp.float32)
    xl_ref[...] = xl
    scal_ref[...] = jnp.dot(xl, am_ref[...], preferred_element_type=jnp.float32)


def _mm(x, Wt, Am, BM=1000):
    n, k = x.shape
    f = Wt.shape[1]
    return pl.pallas_call(
        _mm_body,
        grid=(n // BM,),
        in_specs=[
            pl.BlockSpec((BM, k), lambda i: (i, 0)),
            pl.BlockSpec((k, f), lambda i: (0, 0)),
            pl.BlockSpec((f, 128), lambda i: (0, 0)),
        ],
        out_specs=[
            pl.BlockSpec((BM, f), lambda i: (i, 0)),
            pl.BlockSpec((BM, 128), lambda i: (i, 0)),
        ],
        out_shape=[
            jax.ShapeDtypeStruct((n, f), jnp.float32),
            jax.ShapeDtypeStruct((n, 128), jnp.float32),
        ],
    )(x, Wt, Am)


def _gat_edge_body(C, xl_hbm, asrc_hbm, adst_hbm, batch_hbm, src_hbm, dst_hbm,
                   z2_hbm, z1_hbm, num_hbm, den_hbm,
                   facc, dacc, src_v, dst_v, ex_v,
                   ridx_v, didx_v, hs_v, hd_v, bs_v, bd_v, as_v, ad_v,
                   rows_v, sem):
    head = lax.axis_index("c")
    tid = lax.axis_index("s")
    ebase = tid * _ET
    rbase = jnp.minimum(tid * _RSTRIPE, N - _RSTRIPE)
    dbase = tid * _DSTRIPE
    iota = lax.iota(jnp.int32, 16)
    zeros16 = jnp.zeros((16,), jnp.float32)
    hc = head * C

    pltpu.sync_copy(src_hbm.at[pl.ds(ebase, _ET)], src_v.at[pl.ds(0, _ET)])
    pltpu.sync_copy(dst_hbm.at[pl.ds(ebase, _ET)], dst_v.at[pl.ds(0, _ET)])
    for j in range((_ETP - _ET) // 16):
        src_v[pl.ds(_ET + 16 * j, 16)] = iota * 0
        dst_v[pl.ds(_ET + 16 * j, 16)] = iota * 0

    hoff = head * _NP
    def _exb(b, c):
        base = b * _EBK
        for j in range(_EBK // 16):
            s16 = src_v[pl.ds(base + 16 * j, 16)]
            d16 = dst_v[pl.ds(base + 16 * j, 16)]
            hs_v[pl.ds(16 * j, 16)] = s16 + hoff
            hd_v[pl.ds(16 * j, 16)] = d16 + hoff
        c1 = pltpu.async_copy(batch_hbm.at[src_v.at[pl.ds(base, _EBK)]], bs_v, sem)
        c2 = pltpu.async_copy(batch_hbm.at[dst_v.at[pl.ds(base, _EBK)]], bd_v, sem)
        c3 = pltpu.async_copy(asrc_hbm.at[hs_v], as_v, sem)
        c4 = pltpu.async_copy(adst_hbm.at[hd_v], ad_v, sem)
        c1.wait(); c2.wait(); c3.wait(); c4.wait()
        for j in range(_EBK // 16):
            a = as_v[pl.ds(16 * j, 16)] + ad_v[pl.ds(16 * j, 16)]
            a = jnp.where(a > 0, a, 0.2 * a)
            eq = bs_v[pl.ds(16 * j, 16)] == bd_v[pl.ds(16 * j, 16)]
            ex_v[pl.ds(base + 16 * j, 16)] = jnp.where(eq, jnp.exp(a), 0.0)
        return c
    lax.fori_loop(0, _NBE, _exb, 0)
    for j in range((_ETP - _ET) // 16):
        ex_v[pl.ds(_ET + 16 * j, 16)] = zeros16

    def _zero_facc():
        for q in range(0, _RSTRIPE, 128):
            ln = min(128, _RSTRIPE - q)
            pltpu.sync_copy(z2_hbm.at[pl.ds(0, ln)], facc.at[pl.ds(rbase + q, ln)])
    _zero_facc()
    pltpu.sync_copy(z1_hbm.at[pl.ds(dbase, _DSTRIPE)], dacc.at[pl.ds(dbase, _DSTRIPE)])
    plsc.subcore_barrier()

    def _bld_idx(b, p, ridx_r, didx_r):
        base = b * _BK
        for j in range(_BK // 16):
            s16 = src_v[pl.ds(base + 16 * j, 16)]
            ridx_r[pl.ds(16 * j, 16)] = s16 * (H * C) + (hc + p)
            didx_r[pl.ds(16 * j, 16)] = dst_v[pl.ds(base + 16 * j, 16)]

    def _scale(b, rows_r):
        base = b * _BK
        for g in range(_BK // 16):
            w16 = ex_v[pl.ds(base + 16 * g, 16)]
            for r in range(16):
                wv = jnp.take(w16, jnp.full((16,), r, jnp.int32))
                i = 16 * g + r
                for j in range(8):
                    rows_r[i, pl.ds(16 * j, 16)] = rows_r[i, pl.ds(16 * j, 16)] * wv

    def _pass(p, c):
        def _blk(b, cc):
            _bld_idx(b, p, ridx_v, didx_v)
            pltpu.async_copy(xl_hbm.at[ridx_v], rows_v, sem).wait()
            _scale(b, rows_v)
            pltpu.sync_copy(rows_v, facc.at[didx_v], add=True)

            @pl.when(p == 0)
            def _():
                pltpu.sync_copy(ex_v.at[pl.ds(b * _BK, _BK)], dacc.at[didx_v], add=True)
            return cc
        lax.fori_loop(0, _ETP // _BK, _blk, 0)
        plsc.subcore_barrier()

        for q in range(0, _RSTRIPE, 128):
            ln = min(128, _RSTRIPE - q)
            pltpu.sync_copy(facc.at[pl.ds(rbase + q, ln)],
                            num_hbm.at[hc + p, pl.ds(rbase + q, ln)])

        @pl.when(p == 0)
        def _():
            pltpu.sync_copy(dacc.at[pl.ds(dbase, _DSTRIPE)],
                            den_hbm.at[pl.ds(head * _NP + dbase, _DSTRIPE)])
        plsc.subcore_barrier()
        _zero_facc()
        plsc.subcore_barrier()
        return c
    lax.fori_loop(0, C, _pass, 0)


def _gat_edge(xl_rows, asrcT, adstT, batch, src, dst, C):
    mesh = plsc.VectorSubcoreMesh(core_axis_name="c", subcore_axis_name="s")
    body = functools.partial(_gat_edge_body, C)
    f = pl.kernel(
        body,
        out_type=[
            jax.ShapeDtypeStruct((H * C, N, 128), jnp.float32),
            jax.ShapeDtypeStruct((H * _NP,), jnp.float32),
        ],
        mesh=mesh,
        scratch_types=[
            pltpu.VMEM_SHARED((N, 128), jnp.float32),
            pltpu.VMEM_SHARED((_NP,), jnp.float32),
            pltpu.VMEM((_EVPAD,), jnp.int32),
            pltpu.VMEM((_EVPAD,), jnp.int32),
            pltpu.VMEM((_EVPAD,), jnp.float32),
            pltpu.VMEM((_BK,), jnp.int32),
            pltpu.VMEM((_BK,), jnp.int32),
            pltpu.VMEM((_EBK,), jnp.int32),
            pltpu.VMEM((_EBK,), jnp.int32),
            pltpu.VMEM((_EBK,), jnp.int32),
            pltpu.VMEM((_EBK,), jnp.int32),
            pltpu.VMEM((_EBK,), jnp.float32),
            pltpu.VMEM((_EBK,), jnp.float32),
            pltpu.VMEM((_BK, 128), jnp.float32),
            pltpu.SemaphoreType.DMA,
        ],
    )
    z2 = jnp.zeros((128, 128), jnp.float32)
    z1 = jnp.zeros((_NP,), jnp.float32)
    return f(xl_rows, asrcT, adstT, batch, src, dst, z2, z1)


def _ep_body(C, OC, relu, num_ref, xl_ref, scal_ref, den_ref, b_ref, out_ref):
    xl = xl_ref[...]
    scal = scal_ref[...]
    a_self = scal[:, 0:2] + scal[:, 2:4]
    a_self = jnp.where(a_self > 0, a_self, 0.2 * a_self)
    exs = jnp.exp(a_self)
    den = jnp.maximum(den_ref[...] + exs, 1e-16)
    acc = None
    for h in range(H):
        numh = jnp.concatenate([num_ref[h * C + c] for c in range(C)], axis=-1)
        xlh = xl[:, h * OC:(h + 1) * OC]
        oh = (numh + exs[:, h:h + 1] * xlh) / den[:, h:h + 1]
        acc = oh if acc is None else acc + oh
    out = acc * (1.0 / H) + b_ref[...]
    if relu:
        out = jnp.maximum(out, 0.0)
    out_ref[...] = out


def _ep(num, xl, scal, denT, b2d, C, OC, relu, BM=1000):
    body = functools.partial(_ep_body, C, OC, relu)
    return pl.pallas_call(
        body,
        grid=(N // BM,),
        in_specs=[
            pl.BlockSpec((H * C, BM, 128), lambda i: (0, i, 0)),
            pl.BlockSpec((BM, H * OC), lambda i: (i, 0)),
            pl.BlockSpec((BM, 128), lambda i: (i, 0)),
            pl.BlockSpec((BM, 2), lambda i: (i, 0)),
            pl.BlockSpec((1, OC), lambda i: (0, 0)),
        ],
        out_specs=pl.BlockSpec((BM, OC), lambda i: (i, 0)),
        out_shape=jax.ShapeDtypeStruct((N, OC), jnp.float32),
    )(num, xl, scal, denT, b2d)


def _pool_body(h_ref, batch_ref, wa_ref, ba_ref, wct_ref, bc_ref, out_ref):
    hv = h_ref[...]
    logits = jnp.sum(hv * wa_ref[...], axis=1, keepdims=True) + ba_ref[0, 0]
    b = batch_ref[...]
    gi = lax.broadcasted_iota(jnp.int32, (1, G), 1)
    oh = b == gi
    lm = jnp.max(jnp.where(oh, logits, jnp.float32(-1e30)), axis=0, keepdims=True)
    e = jnp.where(oh, jnp.exp(logits - lm), 0.0)
    s = jnp.sum(e, axis=0, keepdims=True)
    attn = e / s
    attended = lax.dot_general(attn, hv, (((0,), (0,)), ((), ())),
                               preferred_element_type=jnp.float32)
    out_ref[...] = jnp.dot(attended, wct_ref[...],
                           preferred_element_type=jnp.float32) + bc_ref[...]


def _pool(h2, batch2d, w_att, b_att, WcT, bc2d):
    return pl.pallas_call(
        _pool_body,
        grid=(1,),
        in_specs=[
            pl.BlockSpec((N, OUT), lambda i: (0, 0)),
            pl.BlockSpec((N, 1), lambda i: (0, 0)),
            pl.BlockSpec((1, OUT), lambda i: (0, 0)),
            pl.BlockSpec((1, 1), lambda i: (0, 0)),
            pl.BlockSpec((OUT, 2), lambda i: (0, 0)),
            pl.BlockSpec((1, 2), lambda i: (0, 0)),
        ],
        out_specs=pl.BlockSpec((G, 2), lambda i: (0, 0)),
        out_shape=jax.ShapeDtypeStruct((G, 2), jnp.float32),
    )(h2, batch2d, w_att, b_att, WcT, bc2d)


def _att_matrix(att_src, att_dst, OC):
    am = jnp.zeros((H * OC, 128), jnp.float32)
    asr = att_src.reshape(H, OC)
    adr = att_dst.reshape(H, OC)
    for h in range(H):
        am = am.at[h * OC:(h + 1) * OC, h].set(asr[h])
        am = am.at[h * OC:(h + 1) * OC, 2 + h].set(adr[h])
    return am


def _padT(a):
    return jnp.zeros((H, _NP), jnp.float32).at[:, :N].set(a.T).reshape(-1)


def kernel(x, edge_index, batch, W1, att_src1, att_dst1, b1,
           W2, att_src2, att_dst2, b2, w_att, b_att, Wc, bc):
    src = edge_index[0].astype(jnp.int32)
    dst = edge_index[1].astype(jnp.int32)
    batch_i = batch.astype(jnp.int32)

    C1 = HID // 128
    C2 = OUT // 128

    xl1, scal1 = _mm(x, W1.T, _att_matrix(att_src1, att_dst1, HID))
    num1, den1 = _gat_edge(xl1.reshape(N * H * C1, 128),
                           _padT(scal1[:, 0:2]), _padT(scal1[:, 2:4]),
                           batch_i, src, dst, C1)
    denT1 = den1.reshape(H, _NP)[:, :N].T
    h = _ep(num1, xl1, scal1, denT1, b1.reshape(1, HID), C1, HID, relu=True)

    xl2, scal2 = _mm(h, W2.T, _att_matrix(att_src2, att_dst2, OUT))
    num2, den2 = _gat_edge(xl2.reshape(N * H * C2, 128),
                           _padT(scal2[:, 0:2]), _padT(scal2[:, 2:4]),
                           batch_i, src, dst, C2)
    denT2 = den2.reshape(H, _NP)[:, :N].T
    h2 = _ep(num2, xl2, scal2, denT2, b2.reshape(1, OUT), C2, OUT, relu=False)

    return _pool(h2, batch_i.reshape(N, 1), w_att, b_att.reshape(1, 1), Wc.T,
                 bc.reshape(1, 2))


# speedup vs baseline: 1.2707x; 1.0186x over previous
import functools

import jax
import jax.numpy as jnp
from jax import lax
from jax.experimental import pallas as pl
from jax.experimental.pallas import tpu as pltpu
from jax.experimental.pallas import tpu_sc as plsc

N = 10000
E = 160000
IN_CH = 256
HID = 512
OUT = 256
H = 2
G = 16

_NUM_TILES = 16
_ET = E // _NUM_TILES
_BK = 128
_EBK = 128
_ETP = ((_ET + _EBK - 1) // _EBK) * _EBK
_NBE = _ETP // _EBK
_NB2 = _ETP // (2 * _BK)
_EVPAD = _ETP
_RSTRIPE = 632
_NP = 10240
_DSTRIPE = _NP // _NUM_TILES


def _mm_body(x_ref, wt_ref, am_ref, xl_ref, scal_ref):
    xb = x_ref[...]
    xl = jnp.dot(xb, wt_ref[...], preferred_element_type=jnp.float32)
    xl_ref[...] = xl
    scal_ref[...] = jnp.dot(xl, am_ref[...], preferred_element_type=jnp.float32)


def _mm(x, Wt, Am, BM=1000):
    n, k = x.shape
    f = Wt.shape[1]
    return pl.pallas_call(
        _mm_body,
        grid=(n // BM,),
        in_specs=[
            pl.BlockSpec((BM, k), lambda i: (i, 0)),
            pl.BlockSpec((k, f), lambda i: (0, 0)),
            pl.BlockSpec((f, 128), lambda i: (0, 0)),
        ],
        out_specs=[
            pl.BlockSpec((BM, f), lambda i: (i, 0)),
            pl.BlockSpec((BM, 128), lambda i: (i, 0)),
        ],
        out_shape=[
            jax.ShapeDtypeStruct((n, f), jnp.float32),
            jax.ShapeDtypeStruct((n, 128), jnp.float32),
        ],
    )(x, Wt, Am)


def _gat_edge_body(C, xl_hbm, asrc_hbm, adst_hbm, batch_hbm, src_hbm, dst_hbm,
                   z2_hbm, z1_hbm, num_hbm, den_hbm,
                   facc, dacc, src_v, dst_v, ex_v,
                   ridx_v, didx_v, hs_v, hd_v, bs_v, bd_v, as_v, ad_v,
                   rows_v, sem):
    head = lax.axis_index("c")
    tid = lax.axis_index("s")
    ebase = tid * _ET
    rbase = jnp.minimum(tid * _RSTRIPE, N - _RSTRIPE)
    dbase = tid * _DSTRIPE
    iota = lax.iota(jnp.int32, 16)
    zeros16 = jnp.zeros((16,), jnp.float32)
    hc = head * C

    pltpu.sync_copy(src_hbm.at[pl.ds(ebase, _ET)], src_v.at[pl.ds(0, _ET)])
    pltpu.sync_copy(dst_hbm.at[pl.ds(ebase, _ET)], dst_v.at[pl.ds(0, _ET)])
    for j in range((_ETP - _ET) // 16):
        src_v[pl.ds(_ET + 16 * j, 16)] = iota * 0
        dst_v[pl.ds(_ET + 16 * j, 16)] = iota * 0

    hoff = head * _NP
    def _exb(b, c):
        base = b * _EBK
        for j in range(_EBK // 16):
            s16 = src_v[pl.ds(base + 16 * j, 16)]
            d16 = dst_v[pl.ds(base + 16 * j, 16)]
            hs_v[pl.ds(16 * j, 16)] = s16 + hoff
            hd_v[pl.ds(16 * j, 16)] = d16 + hoff
        c1 = pltpu.async_copy(batch_hbm.at[src_v.at[pl.ds(base, _EBK)]], bs_v, sem)
        c2 = pltpu.async_copy(batch_hbm.at[dst_v.at[pl.ds(base, _EBK)]], bd_v, sem)
        c3 = pltpu.async_copy(asrc_hbm.at[hs_v], as_v, sem)
        c4 = pltpu.async_copy(adst_hbm.at[hd_v], ad_v, sem)
        c1.wait(); c2.wait(); c3.wait(); c4.wait()
        for j in range(_EBK // 16):
            a = as_v[pl.ds(16 * j, 16)] + ad_v[pl.ds(16 * j, 16)]
            a = jnp.where(a > 0, a, 0.2 * a)
            eq = bs_v[pl.ds(16 * j, 16)] == bd_v[pl.ds(16 * j, 16)]
            ex_v[pl.ds(base + 16 * j, 16)] = jnp.where(eq, jnp.exp(a), 0.0)
        return c
    lax.fori_loop(0, _NBE, _exb, 0)
    for j in range((_ETP - _ET) // 16):
        ex_v[pl.ds(_ET + 16 * j, 16)] = zeros16

    def _zero_facc():
        for q in range(0, _RSTRIPE, 128):
            ln = min(128, _RSTRIPE - q)
            pltpu.sync_copy(z2_hbm.at[pl.ds(0, ln)], facc.at[pl.ds(rbase + q, ln)])
    _zero_facc()
    pltpu.sync_copy(z1_hbm.at[pl.ds(dbase, _DSTRIPE)], dacc.at[pl.ds(dbase, _DSTRIPE)])
    plsc.subcore_barrier()

    def _bld_idx(b, p, ridx_r, didx_r):
        base = b * _BK
        for j in range(_BK // 16):
            s16 = src_v[pl.ds(base + 16 * j, 16)]
            ridx_r[pl.ds(16 * j, 16)] = s16 * (H * C) + (hc + p)
            didx_r[pl.ds(16 * j, 16)] = dst_v[pl.ds(base + 16 * j, 16)]

    def _scale(b, rows_r):
        base = b * _BK
        for g in range(_BK // 16):
            w16 = ex_v[pl.ds(base + 16 * g, 16)]
            for r in range(16):
                wv = jnp.take(w16, jnp.full((16,), r, jnp.int32))
                i = 16 * g + r
                for j in range(8):
                    rows_r[i, pl.ds(16 * j, 16)] = rows_r[i, pl.ds(16 * j, 16)] * wv

    def _pass(p, c):
        def _blk(b, cc):
            _bld_idx(b, p, ridx_v, didx_v)
            pltpu.async_copy(xl_hbm.at[ridx_v], rows_v, sem).wait()
            _scale(b, rows_v)
            pltpu.sync_copy(rows_v, facc.at[didx_v], add=True)

            @pl.when(p == 0)
            def _():
                pltpu.sync_copy(ex_v.at[pl.ds(b * _BK, _BK)], dacc.at[didx_v], add=True)
            return cc
        lax.fori_loop(0, _ETP // _BK, _blk, 0)
        plsc.subcore_barrier()

        for q in range(0, _RSTRIPE, 128):
            ln = min(128, _RSTRIPE - q)
            pltpu.sync_copy(facc.at[pl.ds(rbase + q, ln)],
                            num_hbm.at[hc + p, pl.ds(rbase + q, ln)])

        @pl.when(p == 0)
        def _():
            pltpu.sync_copy(dacc.at[pl.ds(dbase, _DSTRIPE)],
                            den_hbm.at[pl.ds(head * _NP + dbase, _DSTRIPE)])
        plsc.subcore_barrier()

        @pl.when(p < C - 1)
        def _():
            _zero_facc()
        plsc.subcore_barrier()
        return c
    lax.fori_loop(0, C, _pass, 0)


def _gat_edge(xl_rows, asrcT, adstT, batch, src, dst, C):
    mesh = plsc.VectorSubcoreMesh(core_axis_name="c", subcore_axis_name="s")
    body = functools.partial(_gat_edge_body, C)
    f = pl.kernel(
        body,
        out_type=[
            jax.ShapeDtypeStruct((H * C, N, 128), jnp.float32),
            jax.ShapeDtypeStruct((H * _NP,), jnp.float32),
        ],
        mesh=mesh,
        scratch_types=[
            pltpu.VMEM_SHARED((N, 128), jnp.float32),
            pltpu.VMEM_SHARED((_NP,), jnp.float32),
            pltpu.VMEM((_EVPAD,), jnp.int32),
            pltpu.VMEM((_EVPAD,), jnp.int32),
            pltpu.VMEM((_EVPAD,), jnp.float32),
            pltpu.VMEM((_BK,), jnp.int32),
            pltpu.VMEM((_BK,), jnp.int32),
            pltpu.VMEM((_EBK,), jnp.int32),
            pltpu.VMEM((_EBK,), jnp.int32),
            pltpu.VMEM((_EBK,), jnp.int32),
            pltpu.VMEM((_EBK,), jnp.int32),
            pltpu.VMEM((_EBK,), jnp.float32),
            pltpu.VMEM((_EBK,), jnp.float32),
            pltpu.VMEM((_BK, 128), jnp.float32),
            pltpu.SemaphoreType.DMA,
        ],
    )
    z2 = jnp.zeros((128, 128), jnp.float32)
    z1 = jnp.zeros((_NP,), jnp.float32)
    return f(xl_rows, asrcT, adstT, batch, src, dst, z2, z1)


def _ep_body(C, OC, relu, num_ref, xl_ref, scal_ref, den_ref, b_ref, out_ref):
    xl = xl_ref[...]
    scal = scal_ref[...]
    a_self = scal[:, 0:2] + scal[:, 2:4]
    a_self = jnp.where(a_self > 0, a_self, 0.2 * a_self)
    exs = jnp.exp(a_self)
    den = jnp.maximum(den_ref[...] + exs, 1e-16)
    acc = None
    for h in range(H):
        numh = jnp.concatenate([num_ref[h * C + c] for c in range(C)], axis=-1)
        xlh = xl[:, h * OC:(h + 1) * OC]
        oh = (numh + exs[:, h:h + 1] * xlh) / den[:, h:h + 1]
        acc = oh if acc is None else acc + oh
    out = acc * (1.0 / H) + b_ref[...]
    if relu:
        out = jnp.maximum(out, 0.0)
    out_ref[...] = out


def _ep(num, xl, scal, denT, b2d, C, OC, relu, BM=1000):
    body = functools.partial(_ep_body, C, OC, relu)
    return pl.pallas_call(
        body,
        grid=(N // BM,),
        in_specs=[
            pl.BlockSpec((H * C, BM, 128), lambda i: (0, i, 0)),
            pl.BlockSpec((BM, H * OC), lambda i: (i, 0)),
            pl.BlockSpec((BM, 128), lambda i: (i, 0)),
            pl.BlockSpec((BM, 2), lambda i: (i, 0)),
            pl.BlockSpec((1, OC), lambda i: (0, 0)),
        ],
        out_specs=pl.BlockSpec((BM, OC), lambda i: (i, 0)),
        out_shape=jax.ShapeDtypeStruct((N, OC), jnp.float32),
    )(num, xl, scal, denT, b2d)


def _epmm_body(C, OC, num_ref, xl_ref, scal_ref, den_ref, b_ref, wt_ref, am_ref,
               xl2_ref, scal2_ref):
    xl = xl_ref[...]
    scal = scal_ref[...]
    a_self = scal[:, 0:2] + scal[:, 2:4]
    a_self = jnp.where(a_self > 0, a_self, 0.2 * a_self)
    exs = jnp.exp(a_self)
    den = jnp.maximum(den_ref[...] + exs, 1e-16)
    acc = None
    for h in range(H):
        numh = jnp.concatenate([num_ref[h * C + c] for c in range(C)], axis=-1)
        xlh = xl[:, h * OC:(h + 1) * OC]
        oh = (numh + exs[:, h:h + 1] * xlh) / den[:, h:h + 1]
        acc = oh if acc is None else acc + oh
    hcur = jnp.maximum(acc * (1.0 / H) + b_ref[...], 0.0)
    xl2 = jnp.dot(hcur, wt_ref[...], preferred_element_type=jnp.float32)
    xl2_ref[...] = xl2
    scal2_ref[...] = jnp.dot(xl2, am_ref[...], preferred_element_type=jnp.float32)


def _epmm(num, xl, scal, denT, b2d, Wt, Am, C, OC, BM=1000):
    body = functools.partial(_epmm_body, C, OC)
    k, f = Wt.shape
    return pl.pallas_call(
        body,
        grid=(N // BM,),
        in_specs=[
            pl.BlockSpec((H * C, BM, 128), lambda i: (0, i, 0)),
            pl.BlockSpec((BM, H * OC), lambda i: (i, 0)),
            pl.BlockSpec((BM, 128), lambda i: (i, 0)),
            pl.BlockSpec((BM, 2), lambda i: (i, 0)),
            pl.BlockSpec((1, OC), lambda i: (0, 0)),
            pl.BlockSpec((k, f), lambda i: (0, 0)),
            pl.BlockSpec((f, 128), lambda i: (0, 0)),
        ],
        out_specs=[
            pl.BlockSpec((BM, f), lambda i: (i, 0)),
            pl.BlockSpec((BM, 128), lambda i: (i, 0)),
        ],
        out_shape=[
            jax.ShapeDtypeStruct((N, f), jnp.float32),
            jax.ShapeDtypeStruct((N, 128), jnp.float32),
        ],
    )(num, xl, scal, denT, b2d, Wt, Am)


def _pool_body(h_ref, batch_ref, wa_ref, ba_ref, wct_ref, bc_ref, out_ref):
    hv = h_ref[...]
    logits = jnp.sum(hv * wa_ref[...], axis=1, keepdims=True) + ba_ref[0, 0]
    b = batch_ref[...]
    gi = lax.broadcasted_iota(jnp.int32, (1, G), 1)
    oh = b == gi
    lm = jnp.max(jnp.where(oh, logits, jnp.float32(-1e30)), axis=0, keepdims=True)
    e = jnp.where(oh, jnp.exp(logits - lm), 0.0)
    s = jnp.sum(e, axis=0, keepdims=True)
    attn = e / s
    attended = lax.dot_general(attn, hv, (((0,), (0,)), ((), ())),
                               preferred_element_type=jnp.float32)
    out_ref[...] = jnp.dot(attended, wct_ref[...],
                           preferred_element_type=jnp.float32) + bc_ref[...]


def _pool(h2, batch2d, w_att, b_att, WcT, bc2d):
    return pl.pallas_call(
        _pool_body,
        grid=(1,),
        in_specs=[
            pl.BlockSpec((N, OUT), lambda i: (0, 0)),
            pl.BlockSpec((N, 1), lambda i: (0, 0)),
            pl.BlockSpec((1, OUT), lambda i: (0, 0)),
            pl.BlockSpec((1, 1), lambda i: (0, 0)),
            pl.BlockSpec((OUT, 2), lambda i: (0, 0)),
            pl.BlockSpec((1, 2), lambda i: (0, 0)),
        ],
        out_specs=pl.BlockSpec((G, 2), lambda i: (0, 0)),
        out_shape=jax.ShapeDtypeStruct((G, 2), jnp.float32),
    )(h2, batch2d, w_att, b_att, WcT, bc2d)


def _att_matrix(att_src, att_dst, OC):
    am = jnp.zeros((H * OC, 128), jnp.float32)
    asr = att_src.reshape(H, OC)
    adr = att_dst.reshape(H, OC)
    for h in range(H):
        am = am.at[h * OC:(h + 1) * OC, h].set(asr[h])
        am = am.at[h * OC:(h + 1) * OC, 2 + h].set(adr[h])
    return am


def _padT(a):
    return jnp.zeros((H, _NP), jnp.float32).at[:, :N].set(a.T).reshape(-1)


def kernel(x, edge_index, batch, W1, att_src1, att_dst1, b1,
           W2, att_src2, att_dst2, b2, w_att, b_att, Wc, bc):
    src = edge_index[0].astype(jnp.int32)
    dst = edge_index[1].astype(jnp.int32)
    batch_i = batch.astype(jnp.int32)

    C1 = HID // 128
    C2 = OUT // 128

    xl1, scal1 = _mm(x, W1.T, _att_matrix(att_src1, att_dst1, HID))
    num1, den1 = _gat_edge(xl1.reshape(N * H * C1, 128),
                           _padT(scal1[:, 0:2]), _padT(scal1[:, 2:4]),
                           batch_i, src, dst, C1)
    denT1 = den1.reshape(H, _NP)[:, :N].T
    xl2, scal2 = _epmm(num1, xl1, scal1, denT1, b1.reshape(1, HID), W2.T,
                       _att_matrix(att_src2, att_dst2, OUT), C1, HID)
    num2, den2 = _gat_edge(xl2.reshape(N * H * C2, 128),
                           _padT(scal2[:, 0:2]), _padT(scal2[:, 2:4]),
                           batch_i, src, dst, C2)
    denT2 = den2.reshape(H, _NP)[:, :N].T
    h2 = _ep(num2, xl2, scal2, denT2, b2.reshape(1, OUT), C2, OUT, relu=False)

    return _pool(h2, batch_i.reshape(N, 1), w_att, b_att.reshape(1, 1), Wc.T,
                 bc.reshape(1, 2))
